# baseline (speedup 1.0000x reference)
import functools

import jax
import jax.numpy as jnp
from jax import lax
from jax.experimental import pallas as pl
from jax.experimental.pallas import tpu as pltpu
from jax.experimental.pallas import tpu_sc as plsc

_N = 1024
_D = 64
_T = 16
_NORM_TEMP = 50.0
_SCALE = 0.1 / 32.0
_KB = 4000
_LOG2E = 1.4426950408889634
_LN2 = 0.6931471805599453

_NW = 32
_PAIRS = _N * _T
_PAIRS_W = _PAIRS // _NW


_GRP = 64
_NGRP = _PAIRS_W // _GRP


def _sc_gather_body(te_hbm, idx_hbm, out_hbm, idx_v, sem):
    wid = lax.axis_index("s") * 2 + lax.axis_index("c")
    base = wid * _PAIRS_W
    pltpu.sync_copy(idx_hbm.at[pl.ds(base, _PAIRS_W)], idx_v)

    def fire(b):
        for q in range(_GRP // 16):
            v16 = idx_v[pl.ds(b * _GRP + q * 16, 16)]
            for u in range(16):
                p = b * _GRP + q * 16 + u
                pltpu.async_copy(te_hbm.at[pl.ds(v16[u], 1)],
                                 out_hbm.at[pl.ds(base + p, 1)], sem)

    def drain(b):
        pltpu.make_async_copy(te_hbm.at[pl.ds(0, _GRP)],
                              out_hbm.at[pl.ds(base + b * _GRP, _GRP)],
                              sem).wait()

    fire(0)

    def gb(b, c):
        fire(b)
        drain(b - 1)
        return c

    lax.fori_loop(1, _NGRP, gb, 0)
    drain(_NGRP - 1)


def _sc_gather(tag_embeddings, idx_flat):
    mesh = plsc.VectorSubcoreMesh(core_axis_name="c", subcore_axis_name="s")
    k = pl.kernel(
        _sc_gather_body,
        out_type=jax.ShapeDtypeStruct((_PAIRS, _D), jnp.float32),
        mesh=mesh,
        scratch_types=[
            pltpu.VMEM((_PAIRS_W,), jnp.int32),
            pltpu.SemaphoreType.DMA,
        ],
    )
    return k(tag_embeddings, idx_flat)


def _dense_body(re_ref, te_ref, g_ref, tags_ref, out_ref, ren_ref, *, n_blocks):
    pid = pl.program_id(0)

    @pl.when(pid == 0)
    def _init():
        re = re_ref[...]
        ss = jnp.sum(re * re, axis=1, keepdims=True)
        inv = (_NORM_TEMP * _LOG2E) * lax.rsqrt(jnp.maximum(ss, 1e-24))
        ren_ref[...] = (re * inv).astype(jnp.bfloat16)
        out_ref[0, 0] = 0.0

    te = te_ref[...]
    ss_t = jnp.sum(te * te, axis=1, keepdims=True)
    te_n = (te * lax.rsqrt(jnp.maximum(ss_t, 1e-24))).astype(jnp.bfloat16)
    s2 = lax.dot_general(ren_ref[...], te_n, (((1,), (1,)), ((), ())),
                         preferred_element_type=jnp.float32)
    out_ref[0, 0] += jnp.sum(jnp.log2(1.0 + jnp.exp2(s2)))

    @pl.when(pid == n_blocks - 1)
    def _finish():
        re = re_ref[...]
        ss_re = jnp.sum(re * re, axis=1, keepdims=True)
        lbl = jnp.zeros((), jnp.float32)
        for t in range(_T):
            gt = g_ref[pl.ds(_N * t, _N), :]
            dt = jnp.sum(gt * re, axis=1, keepdims=True)
            st = jnp.sum(gt * gt, axis=1, keepdims=True)
            c = dt * lax.rsqrt(jnp.maximum(st * ss_re, 1e-30))
            tt = tags_ref[:, t:t + 1]
            dup = jnp.zeros((_N, 1), jnp.bool_)
            for k in range(t):
                dup = dup | (tt == tags_ref[:, k:k + 1])
            lbl = lbl + jnp.sum(jnp.where(dup, 0.0, c))
        out_ref[0, 0] = (out_ref[0, 0] * _LN2 - _NORM_TEMP * lbl) * _SCALE


def _dense_loss(region_embeddings, tag_embeddings, g, tags):
    n_blocks = tag_embeddings.shape[0] // _KB
    out = pl.pallas_call(
        functools.partial(_dense_body, n_blocks=n_blocks),
        grid=(n_blocks,),
        in_specs=[
            pl.BlockSpec((_N, _D), lambda i: (0, 0)),
            pl.BlockSpec((_KB, _D), lambda i: (i, 0)),
            pl.BlockSpec((_PAIRS, _D), lambda i: (0, 0)),
            pl.BlockSpec((_N, _T), lambda i: (0, 0)),
        ],
        out_specs=pl.BlockSpec(memory_space=pltpu.SMEM),
        out_shape=jax.ShapeDtypeStruct((1, 1), jnp.float32),
        scratch_shapes=[pltpu.VMEM((_N, _D), jnp.bfloat16)],
        compiler_params=pltpu.CompilerParams(
            dimension_semantics=("arbitrary",),
        ),
    )(region_embeddings, tag_embeddings, g, tags)
    return out[0, 0]


def kernel(region_embeddings, tag_embeddings, tags):
    idx_flat = tags.T.reshape(-1)
    g = _sc_gather(tag_embeddings, idx_flat)
    return _dense_loss(region_embeddings, tag_embeddings, g, tags)


# speedup vs baseline: 1.6405x; 1.6405x over previous
import functools

import jax
import jax.numpy as jnp
from jax import lax
from jax.experimental import pallas as pl
from jax.experimental.pallas import tpu as pltpu
from jax.experimental.pallas import tpu_sc as plsc

_N = 1024
_D = 64
_T = 16
_NORM_TEMP = 50.0
_SCALE = 0.1 / 32.0
_KB = 4000
_LOG2E = 1.4426950408889634
_LN2 = 0.6931471805599453

_NW = 32
_PAIRS = _N * _T
_PAIRS_W = _PAIRS // _NW


def _sc_gather_body(te2_hbm, idx_hbm, out_hbm, idx_v, rows_v, sem):
    wid = lax.axis_index("s") * 2 + lax.axis_index("c")
    base = wid * _PAIRS_W
    pltpu.sync_copy(idx_hbm.at[pl.ds(base, _PAIRS_W)], idx_v)
    pltpu.async_copy(te2_hbm.at[idx_v], rows_v, sem).wait()
    pltpu.sync_copy(rows_v, out_hbm.at[pl.ds(base, _PAIRS_W)])


def _sc_gather(te2, idx_half):
    mesh = plsc.VectorSubcoreMesh(core_axis_name="c", subcore_axis_name="s")
    k = pl.kernel(
        _sc_gather_body,
        out_type=jax.ShapeDtypeStruct((_PAIRS, 2 * _D), jnp.float32),
        mesh=mesh,
        scratch_types=[
            pltpu.VMEM((_PAIRS_W,), jnp.int32),
            pltpu.VMEM((_PAIRS_W, 2 * _D), jnp.float32),
            pltpu.SemaphoreType.DMA,
        ],
    )
    return k(te2, idx_half)


def _dense_body(re_ref, te_ref, g_ref, par_ref, m_ref, out_ref, ren_ref, *, n_blocks):
    pid = pl.program_id(0)

    @pl.when(pid == 0)
    def _init():
        re = re_ref[...]
        ss = jnp.sum(re * re, axis=1, keepdims=True)
        inv = (_NORM_TEMP * _LOG2E) * lax.rsqrt(jnp.maximum(ss, 1e-24))
        ren_ref[...] = (re * inv).astype(jnp.bfloat16)
        out_ref[0, 0] = 0.0

    te = te_ref[...]
    ss_t = jnp.sum(te * te, axis=1, keepdims=True)
    te_n = (te * lax.rsqrt(jnp.maximum(ss_t, 1e-24))).astype(jnp.bfloat16)
    s2 = lax.dot_general(ren_ref[...], te_n, (((1,), (1,)), ((), ())),
                         preferred_element_type=jnp.float32)
    out_ref[0, 0] += jnp.sum(jnp.log2(1.0 + jnp.exp2(s2)))

    @pl.when(pid == n_blocks - 1)
    def _finish():
        re = re_ref[...]
        ss_re = jnp.sum(re * re, axis=1, keepdims=True)
        lbl = jnp.zeros((), jnp.float32)
        for t in range(_T):
            gp = g_ref[pl.ds(_N * t, _N), :]
            odd = par_ref[:, t:t + 1] == 1
            gt = jnp.where(odd, gp[:, _D:], gp[:, :_D])
            dt = jnp.sum(gt * re, axis=1, keepdims=True)
            st = jnp.sum(gt * gt, axis=1, keepdims=True)
            c = dt * lax.rsqrt(jnp.maximum(st * ss_re, 1e-30))
            lbl = lbl + jnp.sum(m_ref[:, t:t + 1] * c)
        out_ref[0, 0] = (out_ref[0, 0] * _LN2 - _NORM_TEMP * lbl) * _SCALE


def _dense_loss(region_embeddings, tag_embeddings, g, parity, mask):
    n_blocks = tag_embeddings.shape[0] // _KB
    out = pl.pallas_call(
        functools.partial(_dense_body, n_blocks=n_blocks),
        grid=(n_blocks,),
        in_specs=[
            pl.BlockSpec((_N, _D), lambda i: (0, 0)),
            pl.BlockSpec((_KB, _D), lambda i: (i, 0)),
            pl.BlockSpec((_PAIRS, 2 * _D), lambda i: (0, 0)),
            pl.BlockSpec((_N, _T), lambda i: (0, 0)),
            pl.BlockSpec((_N, _T), lambda i: (0, 0)),
        ],
        out_specs=pl.BlockSpec(memory_space=pltpu.SMEM),
        out_shape=jax.ShapeDtypeStruct((1, 1), jnp.float32),
        scratch_shapes=[pltpu.VMEM((_N, _D), jnp.bfloat16)],
        compiler_params=pltpu.CompilerParams(
            dimension_semantics=("arbitrary",),
        ),
    )(region_embeddings, tag_embeddings, g, parity, mask)
    return out[0, 0]


def kernel(region_embeddings, tag_embeddings, tags):
    idx_flat = tags.T.reshape(-1)
    idx_half = lax.shift_right_logical(idx_flat, 1)
    parity = (tags & 1).astype(jnp.int32)
    t = jnp.arange(_T)
    eq = (tags[:, :, None] == tags[:, None, :]) & (t[None, None, :] < t[None, :, None])
    mask = jnp.where(jnp.any(eq, axis=-1), 0.0, 1.0).astype(jnp.float32)
    te2 = tag_embeddings.reshape(-1, 2 * _D)
    g = _sc_gather(te2, idx_half)
    return _dense_loss(region_embeddings, tag_embeddings, g, parity, mask)
